# initial kernel scaffold (unmeasured)
import jax
import jax.numpy as jnp
from jax import lax
from jax.experimental import pallas as pl
from jax.experimental.pallas import tpu as pltpu


def kernel(
    x,
):
    def body(*refs):
        pass

    out_shape = jax.ShapeDtypeStruct(..., jnp.float32)
    return pl.pallas_call(body, out_shape=out_shape)(...)


# baseline (device time: 105844 ns/iter reference)
import jax
import jax.numpy as jnp
from jax import lax
from jax.experimental import pallas as pl
from jax.experimental.pallas import tpu as pltpu

N_X = 2


def kernel(x):
    m, n = x.shape

    def body(x_ref, out_ref, comm_ref, send_sem, recv_sem):
        my_x = lax.axis_index("x")
        my_y = lax.axis_index("y")
        my_z = lax.axis_index("z")
        partner = (1 - my_x, my_y, my_z)

        barrier_sem = pltpu.get_barrier_semaphore()
        pl.semaphore_signal(
            barrier_sem, inc=1,
            device_id=partner, device_id_type=pl.DeviceIdType.MESH,
        )
        pl.semaphore_wait(barrier_sem, 1)

        comm_ref[...] = x_ref[...].astype(jnp.bfloat16)

        rdma = pltpu.make_async_remote_copy(
            src_ref=comm_ref,
            dst_ref=out_ref.at[pl.ds(my_x * m, m), :],
            send_sem=send_sem,
            recv_sem=recv_sem,
            device_id=partner,
            device_id_type=pl.DeviceIdType.MESH,
        )
        rdma.start()

        out_ref[pl.ds(my_x * m, m), :] = comm_ref[...]

        rdma.wait()

    return pl.pallas_call(
        body,
        out_shape=jax.ShapeDtypeStruct((N_X * m, n), jnp.bfloat16),
        in_specs=[pl.BlockSpec(memory_space=pltpu.VMEM)],
        out_specs=pl.BlockSpec(memory_space=pltpu.VMEM),
        scratch_shapes=[
            pltpu.VMEM((m, n), jnp.bfloat16),
            pltpu.SemaphoreType.DMA,
            pltpu.SemaphoreType.DMA,
        ],
        compiler_params=pltpu.CompilerParams(collective_id=0),
    )(x)


# device time: 69767 ns/iter; 1.5171x vs baseline; 1.5171x over previous
import jax
import jax.numpy as jnp
from jax import lax
from jax.experimental import pallas as pl
from jax.experimental.pallas import tpu as pltpu

N_X = 2
RING = [
    (0, 0), (0, 1), (0, 2), (0, 3),
    (1, 3), (1, 2), (1, 1), (2, 1),
    (2, 2), (2, 3), (3, 3), (3, 2),
    (3, 1), (3, 0), (2, 0), (1, 0),
]
NR = len(RING)
J = 8


def kernel(x):
    m, n = x.shape
    half = m // 2
    sub = half // J

    def body(x_ref, out_ref, send_x, recv_x, send_f, recv_f):
        my_x = lax.axis_index("x")
        my_y = lax.axis_index("y")
        my_z = lax.axis_index("z")

        def nbr(shift):
            y = 0
            z = 0
            for i, (yy, zz) in enumerate(RING):
                here = (my_y == yy) & (my_z == zz)
                y2, z2 = RING[(i + shift) % NR]
                y = y + here * y2
                z = z + here * z2
            return y, z

        yR, zR = nbr(1)
        yL, zL = nbr(-1)
        p = (my_y + my_z) % 2
        partner = (1 - my_x, my_y, my_z)
        left = (my_x, yL, zL)
        right = (my_x, yR, zR)

        barrier = pltpu.get_barrier_semaphore()
        for dev in (partner, left, right):
            pl.semaphore_signal(
                barrier, inc=1,
                device_id=dev, device_id_type=pl.DeviceIdType.MESH,
            )
        pl.semaphore_wait(barrier, 3)

        my_base = my_x * m
        opp_base = (1 - my_x) * m
        pblk = p * half
        oblk = (1 - p) * half

        out_ref[pl.ds(my_base + pblk, half), :] = (
            x_ref[pl.ds(pblk, half), :].astype(jnp.bfloat16)
        )

        x_rdmas = []
        for j in range(J):
            off = my_base + pblk + j * sub
            r = pltpu.make_async_remote_copy(
                src_ref=out_ref.at[pl.ds(off, sub), :],
                dst_ref=out_ref.at[pl.ds(off, sub), :],
                send_sem=send_x.at[j],
                recv_sem=recv_x.at[j],
                device_id=partner,
                device_id_type=pl.DeviceIdType.MESH,
            )
            r.start()
            x_rdmas.append(r)

        out_ref[pl.ds(my_base + oblk, half), :] = (
            x_ref[pl.ds(oblk, half), :].astype(jnp.bfloat16)
        )

        fwd = []
        for j in range(J):
            off = opp_base + pblk + j * sub
            recv_j = pltpu.make_async_remote_copy(
                src_ref=out_ref.at[pl.ds(off, sub), :],
                dst_ref=out_ref.at[pl.ds(off, sub), :],
                send_sem=send_x.at[j],
                recv_sem=recv_x.at[j],
                device_id=partner,
                device_id_type=pl.DeviceIdType.MESH,
            )
            recv_j.wait_recv()
            f = pltpu.make_async_remote_copy(
                src_ref=out_ref.at[pl.ds(off, sub), :],
                dst_ref=out_ref.at[pl.ds(off, sub), :],
                send_sem=send_f.at[j],
                recv_sem=recv_f.at[j],
                device_id=left if j < J // 2 else right,
                device_id_type=pl.DeviceIdType.MESH,
            )
            f.start()
            fwd.append(f)

        for j in range(J):
            off = opp_base + oblk + j * sub
            rf = pltpu.make_async_remote_copy(
                src_ref=out_ref.at[pl.ds(off, sub), :],
                dst_ref=out_ref.at[pl.ds(off, sub), :],
                send_sem=send_f.at[j],
                recv_sem=recv_f.at[j],
                device_id=right if j < J // 2 else left,
                device_id_type=pl.DeviceIdType.MESH,
            )
            rf.wait_recv()

        for j in range(J):
            x_rdmas[j].wait_send()
            fwd[j].wait_send()

    return pl.pallas_call(
        body,
        out_shape=jax.ShapeDtypeStruct((N_X * m, n), jnp.bfloat16),
        in_specs=[pl.BlockSpec(memory_space=pltpu.VMEM)],
        out_specs=pl.BlockSpec(memory_space=pltpu.VMEM),
        scratch_shapes=[
            pltpu.SemaphoreType.DMA((J,)),
            pltpu.SemaphoreType.DMA((J,)),
            pltpu.SemaphoreType.DMA((J,)),
            pltpu.SemaphoreType.DMA((J,)),
        ],
        compiler_params=pltpu.CompilerParams(collective_id=0),
    )(x)


# device time: 53054 ns/iter; 1.9950x vs baseline; 1.3150x over previous
import jax
import jax.numpy as jnp
from jax import lax
from jax.experimental import pallas as pl
from jax.experimental.pallas import tpu as pltpu

N_X = 2
RING = [
    (0, 0), (0, 1), (0, 2), (0, 3),
    (1, 3), (1, 2), (1, 1), (2, 1),
    (2, 2), (2, 3), (3, 3), (3, 2),
    (3, 1), (3, 0), (2, 0), (1, 0),
]
NR = len(RING)
Js = 8
H2 = 3
XD = Js - 2 * H2


def kernel(x):
    m, n = x.shape
    blk = m // 4
    sub = blk // Js

    def body(x_ref, out_ref,
             send_xq, recv_xq, send_xd, recv_xd,
             send_1cw, recv_1cw, send_1ccw, recv_1ccw,
             send_2cw, recv_2cw, send_2ccw, recv_2ccw):
        my_x = lax.axis_index("x")
        my_y = lax.axis_index("y")
        my_z = lax.axis_index("z")

        r = 0
        for i, (yy, zz) in enumerate(RING):
            r = r + ((my_y == yy) & (my_z == zz)) * i

        def nbr(shift):
            y = 0
            z = 0
            for i, (yy, zz) in enumerate(RING):
                here = (my_y == yy) & (my_z == zz)
                y2, z2 = RING[(i + shift) % NR]
                y = y + here * y2
                z = z + here * z2
            return y, z

        yR, zR = nbr(1)
        yL, zL = nbr(-1)
        partner = (1 - my_x, my_y, my_z)
        left = (my_x, yL, zL)
        right = (my_x, yR, zR)

        q = r % 4
        c1 = (q + 1) % 4
        c2 = (q + 2) % 4
        c3 = (q + 3) % 4

        barrier = pltpu.get_barrier_semaphore()
        for dev in (partner, left, right):
            pl.semaphore_signal(
                barrier, inc=1,
                device_id=dev, device_id_type=pl.DeviceIdType.MESH,
            )
        pl.semaphore_wait(barrier, 3)

        my_base = my_x * m
        opp_base = (1 - my_x) * m

        def conv(c):
            out_ref[pl.ds(my_base + c * blk, blk), :] = (
                x_ref[pl.ds(c * blk, blk), :].astype(jnp.bfloat16)
            )

        def send(src_off, dst_off, dev, ssem, rsem):
            rd = pltpu.make_async_remote_copy(
                src_ref=out_ref.at[pl.ds(src_off, sub), :],
                dst_ref=out_ref.at[pl.ds(dst_off, sub), :],
                send_sem=ssem, recv_sem=rsem,
                device_id=dev, device_id_type=pl.DeviceIdType.MESH,
            )
            rd.start()
            return rd

        def wait_recv(dst_off, rsem):
            rd = pltpu.make_async_remote_copy(
                src_ref=out_ref.at[pl.ds(dst_off, sub), :],
                dst_ref=out_ref.at[pl.ds(dst_off, sub), :],
                send_sem=send_xq.at[0],
                recv_sem=rsem,
                device_id=partner,
                device_id_type=pl.DeviceIdType.MESH,
            )
            rd.wait_recv()

        started = []

        conv(q)
        for j in range(Js):
            off = my_base + q * blk + j * sub
            started.append(
                send(off, off, partner, send_xq.at[j], recv_xq.at[j]))

        conv(c2)
        for k in range(XD):
            off = my_base + c2 * blk + (H2 + k) * sub
            started.append(
                send(off, off, partner, send_xd.at[k], recv_xd.at[k]))

        conv(c1)
        conv(c3)

        for j in range(Js):
            off = opp_base + q * blk + j * sub
            wait_recv(off, recv_xq.at[j])
            started.append(
                send(off, off, right, send_1cw.at[j], recv_1cw.at[j]))
            started.append(
                send(off, off, left, send_1ccw.at[j], recv_1ccw.at[j]))

        for k in range(H2):
            off = opp_base + c3 * blk + k * sub
            wait_recv(off, recv_1cw.at[k])
            started.append(
                send(off, off, right, send_2cw.at[k], recv_2cw.at[k]))

        for k in range(H2):
            jj = Js - H2 + k
            off = opp_base + c1 * blk + jj * sub
            wait_recv(off, recv_1ccw.at[jj])
            started.append(
                send(off, off, left, send_2ccw.at[k], recv_2ccw.at[k]))

        for j in range(H2, Js):
            wait_recv(opp_base + c3 * blk + j * sub, recv_1cw.at[j])
        for j in range(Js - H2):
            wait_recv(opp_base + c1 * blk + j * sub, recv_1ccw.at[j])
        for k in range(XD):
            wait_recv(opp_base + c2 * blk + (H2 + k) * sub, recv_xd.at[k])
        for k in range(H2):
            wait_recv(opp_base + c2 * blk + k * sub, recv_2cw.at[k])
        for k in range(H2):
            wait_recv(opp_base + c2 * blk + (Js - H2 + k) * sub,
                      recv_2ccw.at[k])

        for rd in started:
            rd.wait_send()

    dma = pltpu.SemaphoreType.DMA
    return pl.pallas_call(
        body,
        out_shape=jax.ShapeDtypeStruct((N_X * m, n), jnp.bfloat16),
        in_specs=[pl.BlockSpec(memory_space=pltpu.VMEM)],
        out_specs=pl.BlockSpec(memory_space=pltpu.VMEM),
        scratch_shapes=[
            dma((Js,)), dma((Js,)),
            dma((max(XD, 1),)), dma((max(XD, 1),)),
            dma((Js,)), dma((Js,)),
            dma((Js,)), dma((Js,)),
            dma((H2,)), dma((H2,)),
            dma((H2,)), dma((H2,)),
        ],
        compiler_params=pltpu.CompilerParams(collective_id=0),
    )(x)


# device time: 52073 ns/iter; 2.0326x vs baseline; 1.0188x over previous
import jax
import jax.numpy as jnp
from jax import lax
from jax.experimental import pallas as pl
from jax.experimental.pallas import tpu as pltpu

N_X = 2
RING = [
    (0, 0), (0, 1), (0, 2), (0, 3),
    (1, 3), (1, 2), (1, 1), (2, 1),
    (2, 2), (2, 3), (3, 3), (3, 2),
    (3, 1), (3, 0), (2, 0), (1, 0),
]
NR = len(RING)
Js = 16
H2 = 6
XD = Js - 2 * H2


def kernel(x):
    m, n = x.shape
    blk = m // 4
    sub = blk // Js

    def body(x_ref, out_ref,
             send_xq, recv_xq, send_xd, recv_xd,
             send_1cw, recv_1cw, send_1ccw, recv_1ccw,
             send_2cw, recv_2cw, send_2ccw, recv_2ccw):
        my_x = lax.axis_index("x")
        my_y = lax.axis_index("y")
        my_z = lax.axis_index("z")

        r = 0
        for i, (yy, zz) in enumerate(RING):
            r = r + ((my_y == yy) & (my_z == zz)) * i

        def nbr(shift):
            y = 0
            z = 0
            for i, (yy, zz) in enumerate(RING):
                here = (my_y == yy) & (my_z == zz)
                y2, z2 = RING[(i + shift) % NR]
                y = y + here * y2
                z = z + here * z2
            return y, z

        yR, zR = nbr(1)
        yL, zL = nbr(-1)
        partner = (1 - my_x, my_y, my_z)
        left = (my_x, yL, zL)
        right = (my_x, yR, zR)

        q = r % 4
        c1 = (q + 1) % 4
        c2 = (q + 2) % 4
        c3 = (q + 3) % 4

        barrier = pltpu.get_barrier_semaphore()
        for dev in (partner, left, right):
            pl.semaphore_signal(
                barrier, inc=1,
                device_id=dev, device_id_type=pl.DeviceIdType.MESH,
            )
        pl.semaphore_wait(barrier, 3)

        my_base = my_x * m
        opp_base = (1 - my_x) * m

        def conv(c):
            out_ref[pl.ds(my_base + c * blk, blk), :] = (
                x_ref[pl.ds(c * blk, blk), :].astype(jnp.bfloat16)
            )

        def send(src_off, dst_off, dev, ssem, rsem):
            rd = pltpu.make_async_remote_copy(
                src_ref=out_ref.at[pl.ds(src_off, sub), :],
                dst_ref=out_ref.at[pl.ds(dst_off, sub), :],
                send_sem=ssem, recv_sem=rsem,
                device_id=dev, device_id_type=pl.DeviceIdType.MESH,
            )
            rd.start()
            return rd

        def wait_recv(dst_off, rsem):
            rd = pltpu.make_async_remote_copy(
                src_ref=out_ref.at[pl.ds(dst_off, sub), :],
                dst_ref=out_ref.at[pl.ds(dst_off, sub), :],
                send_sem=send_xq.at[0],
                recv_sem=rsem,
                device_id=partner,
                device_id_type=pl.DeviceIdType.MESH,
            )
            rd.wait_recv()

        started = []

        for j in range(Js):
            off = my_base + q * blk + j * sub
            out_ref[pl.ds(off, sub), :] = (
                x_ref[pl.ds(q * blk + j * sub, sub), :].astype(jnp.bfloat16)
            )
            started.append(
                send(off, off, partner, send_xq.at[j], recv_xq.at[j]))

        conv(c2)
        for k in range(XD):
            off = my_base + c2 * blk + (H2 + k) * sub
            started.append(
                send(off, off, partner, send_xd.at[k], recv_xd.at[k]))

        conv(c1)
        conv(c3)

        for j in range(Js):
            off = opp_base + q * blk + j * sub
            wait_recv(off, recv_xq.at[j])
            started.append(
                send(off, off, right, send_1cw.at[j], recv_1cw.at[j]))
            started.append(
                send(off, off, left, send_1ccw.at[j], recv_1ccw.at[j]))

        for k in range(H2):
            off = opp_base + c3 * blk + k * sub
            wait_recv(off, recv_1cw.at[k])
            started.append(
                send(off, off, right, send_2cw.at[k], recv_2cw.at[k]))

        for k in range(H2):
            jj = Js - H2 + k
            off = opp_base + c1 * blk + jj * sub
            wait_recv(off, recv_1ccw.at[jj])
            started.append(
                send(off, off, left, send_2ccw.at[k], recv_2ccw.at[k]))

        for j in range(H2, Js):
            wait_recv(opp_base + c3 * blk + j * sub, recv_1cw.at[j])
        for j in range(Js - H2):
            wait_recv(opp_base + c1 * blk + j * sub, recv_1ccw.at[j])
        for k in range(XD):
            wait_recv(opp_base + c2 * blk + (H2 + k) * sub, recv_xd.at[k])
        for k in range(H2):
            wait_recv(opp_base + c2 * blk + k * sub, recv_2cw.at[k])
        for k in range(H2):
            wait_recv(opp_base + c2 * blk + (Js - H2 + k) * sub,
                      recv_2ccw.at[k])

        for rd in started:
            rd.wait_send()

    dma = pltpu.SemaphoreType.DMA
    return pl.pallas_call(
        body,
        out_shape=jax.ShapeDtypeStruct((N_X * m, n), jnp.bfloat16),
        in_specs=[pl.BlockSpec(memory_space=pltpu.VMEM)],
        out_specs=pl.BlockSpec(memory_space=pltpu.VMEM),
        scratch_shapes=[
            dma((Js,)), dma((Js,)),
            dma((max(XD, 1),)), dma((max(XD, 1),)),
            dma((Js,)), dma((Js,)),
            dma((Js,)), dma((Js,)),
            dma((H2,)), dma((H2,)),
            dma((H2,)), dma((H2,)),
        ],
        compiler_params=pltpu.CompilerParams(collective_id=0),
    )(x)


# device time: 50898 ns/iter; 2.0795x vs baseline; 1.0231x over previous
import jax
import jax.numpy as jnp
from jax import lax
from jax.experimental import pallas as pl
from jax.experimental.pallas import tpu as pltpu

N_X = 2
RING = [
    (0, 0), (0, 1), (0, 2), (0, 3),
    (1, 3), (1, 2), (1, 1), (2, 1),
    (2, 2), (2, 3), (3, 3), (3, 2),
    (3, 1), (3, 0), (2, 0), (1, 0),
]
NR = len(RING)
Js = 16
H2 = 6
XD = Js - 2 * H2


def kernel(x):
    m, n = x.shape
    blk = m // 4
    sub = blk // Js

    def body(x_ref, out_ref, scr, stage_q, stage_cw, stage_ccw,
             send_xq, recv_xq, send_xd, recv_xd,
             send_1cw, recv_1cw, send_1ccw, recv_1ccw,
             send_2cw, recv_2cw, send_2ccw, recv_2ccw,
             loc_sems):
        my_x = lax.axis_index("x")
        my_y = lax.axis_index("y")
        my_z = lax.axis_index("z")

        r = 0
        for i, (yy, zz) in enumerate(RING):
            r = r + ((my_y == yy) & (my_z == zz)) * i

        def nbr(shift):
            y = 0
            z = 0
            for i, (yy, zz) in enumerate(RING):
                here = (my_y == yy) & (my_z == zz)
                y2, z2 = RING[(i + shift) % NR]
                y = y + here * y2
                z = z + here * z2
            return y, z

        yR, zR = nbr(1)
        yL, zL = nbr(-1)
        partner = (1 - my_x, my_y, my_z)
        left = (my_x, yL, zL)
        right = (my_x, yR, zR)

        q = r % 4
        c1 = (q + 1) % 4
        c2 = (q + 2) % 4
        c3 = (q + 3) % 4

        barrier = pltpu.get_barrier_semaphore()
        for dev in (partner, left, right):
            pl.semaphore_signal(
                barrier, inc=1,
                device_id=dev, device_id_type=pl.DeviceIdType.MESH,
            )
        pl.semaphore_wait(barrier, 3)

        my_base = my_x * m
        opp_base = (1 - my_x) * m

        def rsend(src_ref, dst_ref, dev, ssem, rsem):
            rd = pltpu.make_async_remote_copy(
                src_ref=src_ref, dst_ref=dst_ref,
                send_sem=ssem, recv_sem=rsem,
                device_id=dev, device_id_type=pl.DeviceIdType.MESH,
            )
            rd.start()
            return rd

        def rwait(dst_ref, rsem):
            rd = pltpu.make_async_remote_copy(
                src_ref=dst_ref, dst_ref=dst_ref,
                send_sem=send_xq.at[0],
                recv_sem=rsem,
                device_id=partner,
                device_id_type=pl.DeviceIdType.MESH,
            )
            rd.wait_recv()

        started = []
        local_dmas = []

        def flush(src_ref, dst_off, rows, sem):
            cp = pltpu.make_async_copy(
                src_ref, out_ref.at[pl.ds(dst_off, rows), :], sem)
            cp.start()
            local_dmas.append(cp)

        for j in range(Js):
            scr[pl.ds(q * blk + j * sub, sub), :] = (
                x_ref[pl.ds(q * blk + j * sub, sub), :].astype(jnp.bfloat16)
            )
            started.append(rsend(
                scr.at[pl.ds(q * blk + j * sub, sub), :],
                stage_q.at[pl.ds(j * sub, sub), :],
                partner, send_xq.at[j], recv_xq.at[j]))

        scr[pl.ds(c2 * blk, blk), :] = (
            x_ref[pl.ds(c2 * blk, blk), :].astype(jnp.bfloat16)
        )
        for k in range(XD):
            soff = c2 * blk + (2 * H2 + k) * sub
            started.append(rsend(
                scr.at[pl.ds(soff, sub), :],
                out_ref.at[pl.ds(my_base + soff, sub), :],
                partner, send_xd.at[k], recv_xd.at[k]))

        scr[pl.ds(c1 * blk, blk), :] = (
            x_ref[pl.ds(c1 * blk, blk), :].astype(jnp.bfloat16)
        )
        scr[pl.ds(c3 * blk, blk), :] = (
            x_ref[pl.ds(c3 * blk, blk), :].astype(jnp.bfloat16)
        )
        flush(scr, my_base, m, loc_sems.at[0])

        for j in range(Js):
            sl = stage_q.at[pl.ds(j * sub, sub), :]
            rwait(sl, recv_xq.at[j])
            started.append(rsend(
                sl, stage_cw.at[pl.ds(j * sub, sub), :],
                right, send_1cw.at[j], recv_1cw.at[j]))
            started.append(rsend(
                sl, stage_ccw.at[pl.ds(j * sub, sub), :],
                left, send_1ccw.at[j], recv_1ccw.at[j]))
        flush(stage_q, opp_base + q * blk, blk, loc_sems.at[1])

        for k in range(H2):
            sl = stage_cw.at[pl.ds(k * sub, sub), :]
            rwait(sl, recv_1cw.at[k])
            started.append(rsend(
                sl, out_ref.at[pl.ds(opp_base + c3 * blk + k * sub, sub), :],
                right, send_2cw.at[k], recv_2cw.at[k]))

        for k in range(H2):
            jj = H2 + k
            sl = stage_ccw.at[pl.ds(jj * sub, sub), :]
            rwait(sl, recv_1ccw.at[jj])
            started.append(rsend(
                sl, out_ref.at[pl.ds(opp_base + c1 * blk + jj * sub, sub), :],
                left, send_2ccw.at[k], recv_2ccw.at[k]))

        for j in range(H2, Js):
            rwait(stage_cw.at[pl.ds(j * sub, sub), :], recv_1cw.at[j])
        flush(stage_cw, opp_base + c3 * blk, blk, loc_sems.at[2])
        for j in list(range(H2)) + list(range(2 * H2, Js)):
            rwait(stage_ccw.at[pl.ds(j * sub, sub), :], recv_1ccw.at[j])
        flush(stage_ccw, opp_base + c1 * blk, blk, loc_sems.at[3])

        for k in range(XD):
            off = opp_base + c2 * blk + (2 * H2 + k) * sub
            rwait(out_ref.at[pl.ds(off, sub), :], recv_xd.at[k])
        for k in range(H2):
            rwait(out_ref.at[pl.ds(opp_base + c2 * blk + k * sub, sub), :],
                  recv_2cw.at[k])
        for k in range(H2):
            off = opp_base + c2 * blk + (H2 + k) * sub
            rwait(out_ref.at[pl.ds(off, sub), :], recv_2ccw.at[k])

        for rd in started:
            rd.wait_send()
        for cp in local_dmas:
            cp.wait()

    dma = pltpu.SemaphoreType.DMA
    return pl.pallas_call(
        body,
        out_shape=jax.ShapeDtypeStruct((N_X * m, n), jnp.bfloat16),
        in_specs=[pl.BlockSpec(memory_space=pltpu.VMEM)],
        out_specs=pl.BlockSpec(memory_space=pl.ANY),
        scratch_shapes=[
            pltpu.VMEM((m, n), jnp.bfloat16),
            pltpu.VMEM((blk, n), jnp.bfloat16),
            pltpu.VMEM((blk, n), jnp.bfloat16),
            pltpu.VMEM((blk, n), jnp.bfloat16),
            dma((Js,)), dma((Js,)),
            dma((max(XD, 1),)), dma((max(XD, 1),)),
            dma((Js,)), dma((Js,)),
            dma((Js,)), dma((Js,)),
            dma((H2,)), dma((H2,)),
            dma((H2,)), dma((H2,)),
            dma((4,)),
        ],
        compiler_params=pltpu.CompilerParams(collective_id=0),
    )(x)


# device time: 50055 ns/iter; 2.1146x vs baseline; 1.0168x over previous
import jax
import jax.numpy as jnp
from jax import lax
from jax.experimental import pallas as pl
from jax.experimental.pallas import tpu as pltpu

N_X = 2
RING = [
    (0, 0), (0, 1), (0, 2), (0, 3),
    (1, 3), (1, 2), (1, 1), (2, 1),
    (2, 2), (2, 3), (3, 3), (3, 2),
    (3, 1), (3, 0), (2, 0), (1, 0),
]
NR = len(RING)
Js = 16
H2 = 6
XD = Js - 2 * H2


def kernel(x):
    m, n = x.shape
    blk = m // 4
    sub = blk // Js

    def body(x_ref, out_ref, scr, stage_q, stage_cw, stage_ccw, xin,
             send_xq, recv_xq, send_xd, recv_xd,
             send_1cw, recv_1cw, send_1ccw, recv_1ccw,
             send_2cw, recv_2cw, send_2ccw, recv_2ccw,
             loc_sems, in_q_sems, in_blk_sems):
        my_x = lax.axis_index("x")
        my_y = lax.axis_index("y")
        my_z = lax.axis_index("z")

        r = 0
        for i, (yy, zz) in enumerate(RING):
            r = r + ((my_y == yy) & (my_z == zz)) * i

        def nbr(shift):
            y = 0
            z = 0
            for i, (yy, zz) in enumerate(RING):
                here = (my_y == yy) & (my_z == zz)
                y2, z2 = RING[(i + shift) % NR]
                y = y + here * y2
                z = z + here * z2
            return y, z

        yR, zR = nbr(1)
        yL, zL = nbr(-1)
        partner = (1 - my_x, my_y, my_z)
        left = (my_x, yL, zL)
        right = (my_x, yR, zR)

        q = r % 4
        c1 = (q + 1) % 4
        c2 = (q + 2) % 4
        c3 = (q + 3) % 4

        barrier = pltpu.get_barrier_semaphore()
        for dev in (partner, left, right):
            pl.semaphore_signal(
                barrier, inc=1,
                device_id=dev, device_id_type=pl.DeviceIdType.MESH,
            )
        pl.semaphore_wait(barrier, 3)

        my_base = my_x * m
        opp_base = (1 - my_x) * m

        def rsend(src_ref, dst_ref, dev, ssem, rsem):
            rd = pltpu.make_async_remote_copy(
                src_ref=src_ref, dst_ref=dst_ref,
                send_sem=ssem, recv_sem=rsem,
                device_id=dev, device_id_type=pl.DeviceIdType.MESH,
            )
            rd.start()
            return rd

        def rwait(dst_ref, rsem):
            rd = pltpu.make_async_remote_copy(
                src_ref=dst_ref, dst_ref=dst_ref,
                send_sem=send_xq.at[0],
                recv_sem=rsem,
                device_id=partner,
                device_id_type=pl.DeviceIdType.MESH,
            )
            rd.wait_recv()

        started = []
        local_dmas = []

        def flush(src_ref, dst_off, rows, sem):
            cp = pltpu.make_async_copy(
                src_ref, out_ref.at[pl.ds(dst_off, rows), :], sem)
            cp.start()
            local_dmas.append(cp)

        in_q = []
        for j in range(Js):
            cp = pltpu.make_async_copy(
                x_ref.at[pl.ds(q * blk + j * sub, sub), :],
                xin.at[0, pl.ds(j * sub, sub), :],
                in_q_sems.at[j])
            cp.start()
            in_q.append(cp)
        cp_c2 = pltpu.make_async_copy(
            x_ref.at[pl.ds(c2 * blk, blk), :], xin.at[1], in_blk_sems.at[0])
        cp_c2.start()

        for j in range(Js):
            in_q[j].wait()
            scr[pl.ds(q * blk + j * sub, sub), :] = (
                xin[0, pl.ds(j * sub, sub), :].astype(jnp.bfloat16)
            )
            started.append(rsend(
                scr.at[pl.ds(q * blk + j * sub, sub), :],
                stage_q.at[pl.ds(j * sub, sub), :],
                partner, send_xq.at[j], recv_xq.at[j]))

        cp_c2.wait()
        cp_c1 = pltpu.make_async_copy(
            x_ref.at[pl.ds(c1 * blk, blk), :], xin.at[0], in_blk_sems.at[1])
        cp_c1.start()
        scr[pl.ds(c2 * blk, blk), :] = xin[1, :, :].astype(jnp.bfloat16)
        for k in range(XD):
            soff = c2 * blk + (2 * H2 + k) * sub
            started.append(rsend(
                scr.at[pl.ds(soff, sub), :],
                out_ref.at[pl.ds(my_base + soff, sub), :],
                partner, send_xd.at[k], recv_xd.at[k]))

        cp_c1.wait()
        cp_c3 = pltpu.make_async_copy(
            x_ref.at[pl.ds(c3 * blk, blk), :], xin.at[1], in_blk_sems.at[2])
        cp_c3.start()
        scr[pl.ds(c1 * blk, blk), :] = xin[0, :, :].astype(jnp.bfloat16)
        cp_c3.wait()
        scr[pl.ds(c3 * blk, blk), :] = xin[1, :, :].astype(jnp.bfloat16)
        flush(scr, my_base, m, loc_sems.at[0])

        for j in range(Js):
            sl = stage_q.at[pl.ds(j * sub, sub), :]
            rwait(sl, recv_xq.at[j])
            started.append(rsend(
                sl, stage_cw.at[pl.ds(j * sub, sub), :],
                right, send_1cw.at[j], recv_1cw.at[j]))
            started.append(rsend(
                sl, stage_ccw.at[pl.ds(j * sub, sub), :],
                left, send_1ccw.at[j], recv_1ccw.at[j]))
        flush(stage_q, opp_base + q * blk, blk, loc_sems.at[1])

        for k in range(H2):
            sl = stage_cw.at[pl.ds(k * sub, sub), :]
            rwait(sl, recv_1cw.at[k])
            started.append(rsend(
                sl, out_ref.at[pl.ds(opp_base + c3 * blk + k * sub, sub), :],
                right, send_2cw.at[k], recv_2cw.at[k]))

        for k in range(H2):
            jj = H2 + k
            sl = stage_ccw.at[pl.ds(jj * sub, sub), :]
            rwait(sl, recv_1ccw.at[jj])
            started.append(rsend(
                sl, out_ref.at[pl.ds(opp_base + c1 * blk + jj * sub, sub), :],
                left, send_2ccw.at[k], recv_2ccw.at[k]))

        for j in range(H2, Js):
            rwait(stage_cw.at[pl.ds(j * sub, sub), :], recv_1cw.at[j])
        flush(stage_cw, opp_base + c3 * blk, blk, loc_sems.at[2])
        for j in list(range(H2)) + list(range(2 * H2, Js)):
            rwait(stage_ccw.at[pl.ds(j * sub, sub), :], recv_1ccw.at[j])
        flush(stage_ccw, opp_base + c1 * blk, blk, loc_sems.at[3])

        for k in range(XD):
            off = opp_base + c2 * blk + (2 * H2 + k) * sub
            rwait(out_ref.at[pl.ds(off, sub), :], recv_xd.at[k])
        for k in range(H2):
            rwait(out_ref.at[pl.ds(opp_base + c2 * blk + k * sub, sub), :],
                  recv_2cw.at[k])
        for k in range(H2):
            off = opp_base + c2 * blk + (H2 + k) * sub
            rwait(out_ref.at[pl.ds(off, sub), :], recv_2ccw.at[k])

        for rd in started:
            rd.wait_send()
        for cp in local_dmas:
            cp.wait()

    dma = pltpu.SemaphoreType.DMA
    return pl.pallas_call(
        body,
        out_shape=jax.ShapeDtypeStruct((N_X * m, n), jnp.bfloat16),
        in_specs=[pl.BlockSpec(memory_space=pl.ANY)],
        out_specs=pl.BlockSpec(memory_space=pl.ANY),
        scratch_shapes=[
            pltpu.VMEM((m, n), jnp.bfloat16),
            pltpu.VMEM((blk, n), jnp.bfloat16),
            pltpu.VMEM((blk, n), jnp.bfloat16),
            pltpu.VMEM((blk, n), jnp.bfloat16),
            pltpu.VMEM((2, blk, n), jnp.float32),
            dma((Js,)), dma((Js,)),
            dma((max(XD, 1),)), dma((max(XD, 1),)),
            dma((Js,)), dma((Js,)),
            dma((Js,)), dma((Js,)),
            dma((H2,)), dma((H2,)),
            dma((H2,)), dma((H2,)),
            dma((4,)),
            dma((Js,)),
            dma((3,)),
        ],
        compiler_params=pltpu.CompilerParams(collective_id=0),
    )(x)
